# Initial kernel scaffold; baseline (speedup 1.0000x reference)
import jax
import jax.numpy as jnp
from jax.experimental import pallas as pl


def kernel(x, table):
    raise NotImplementedError("write your pallas kernel here")


# speedup vs baseline: 6.3322x; 6.3322x over previous
import functools

import jax
import jax.numpy as jnp
from jax import lax
from jax.experimental import pallas as pl
from jax.experimental.pallas import tpu as pltpu
from jax.experimental.pallas import tpu_sc as plsc

DIM = 32
CHUNK = 2048


@functools.partial(jax.jit, static_argnames=("n_rows",))
def _sc_gather(table, idx_flat, n_rows):
    info = plsc.get_sparse_core_info()
    nw = info.num_cores * info.num_subcores
    per_w = n_rows // nw
    n_chunks = per_w // CHUNK
    mesh = plsc.VectorSubcoreMesh(core_axis_name="c", subcore_axis_name="s")

    @functools.partial(
        pl.kernel,
        mesh=mesh,
        out_type=jax.ShapeDtypeStruct((n_rows, DIM), jnp.float32),
        scratch_types=[
            pltpu.VMEM((CHUNK,), jnp.int32),
            pltpu.VMEM((CHUNK, DIM), jnp.float32),
            pltpu.SemaphoreType.DMA,
        ],
        compiler_params=pltpu.CompilerParams(use_tc_tiling_on_sc=False),
    )
    def k(table_hbm, idx_hbm, out_hbm, idx_v, rows_v, sem):
        wid = lax.axis_index("s") * info.num_cores + lax.axis_index("c")
        base = wid * per_w

        def body(c, carry):
            off = base + c * CHUNK
            pltpu.sync_copy(idx_hbm.at[pl.ds(off, CHUNK)], idx_v)
            pltpu.async_copy(table_hbm.at[idx_v], rows_v, sem).wait()
            pltpu.sync_copy(rows_v, out_hbm.at[pl.ds(off, CHUNK), :])
            return carry

        lax.fori_loop(0, n_chunks, body, 0)

    return k(table, idx_flat)


def kernel(x, table):
    n_rows = x.shape[0] * x.shape[1]
    out = _sc_gather(table, x.reshape(n_rows), n_rows)
    return out.reshape(x.shape + (DIM,))


# speedup vs baseline: 6.4530x; 1.0191x over previous
import functools

import jax
import jax.numpy as jnp
from jax import lax
from jax.experimental import pallas as pl
from jax.experimental.pallas import tpu as pltpu
from jax.experimental.pallas import tpu_sc as plsc

DIM = 32
CHUNK = 1600


@functools.partial(jax.jit, static_argnames=("n_rows",))
def _sc_gather(table, idx_flat, n_rows):
    info = plsc.get_sparse_core_info()
    nw = info.num_cores * info.num_subcores
    per_w = n_rows // nw
    n_chunks = per_w // CHUNK
    n_pairs = n_chunks // 2
    mesh = plsc.VectorSubcoreMesh(core_axis_name="c", subcore_axis_name="s")

    @functools.partial(
        pl.kernel,
        mesh=mesh,
        out_type=jax.ShapeDtypeStruct((n_rows, DIM), jnp.float32),
        scratch_types=[
            pltpu.VMEM((CHUNK,), jnp.int32),
            pltpu.VMEM((CHUNK,), jnp.int32),
            pltpu.VMEM((CHUNK, DIM), jnp.float32),
            pltpu.VMEM((CHUNK, DIM), jnp.float32),
            pltpu.SemaphoreType.DMA,
            pltpu.SemaphoreType.DMA,
            pltpu.SemaphoreType.DMA,
            pltpu.SemaphoreType.DMA,
            pltpu.SemaphoreType.DMA,
            pltpu.SemaphoreType.DMA,
        ],
        compiler_params=pltpu.CompilerParams(use_tc_tiling_on_sc=False),
    )
    def k(table_hbm, idx_hbm, out_hbm, idx0, idx1, rows0, rows1,
          isem0, isem1, gsem0, gsem1, osem0, osem1):
        wid = lax.axis_index("s") * info.num_cores + lax.axis_index("c")
        base = wid * per_w

        def idx_start(c, idx_v, isem):
            pltpu.async_copy(idx_hbm.at[pl.ds(base + c * CHUNK, CHUNK)],
                             idx_v, isem)

        def idx_wait(idx_v, isem):
            pltpu.make_async_copy(idx_hbm.at[pl.ds(base, CHUNK)],
                                  idx_v, isem).wait()

        def gat_start(idx_v, rows_v, gsem):
            pltpu.async_copy(table_hbm.at[idx_v], rows_v, gsem)

        def gat_wait(idx_v, rows_v, gsem):
            pltpu.make_async_copy(table_hbm.at[idx_v], rows_v, gsem).wait()

        def out_start(c, rows_v, osem):
            pltpu.async_copy(rows_v,
                             out_hbm.at[pl.ds(base + c * CHUNK, CHUNK), :],
                             osem)

        def out_wait(rows_v, osem):
            pltpu.make_async_copy(rows_v,
                                  out_hbm.at[pl.ds(base, CHUNK), :],
                                  osem).wait()

        idx_start(0, idx0, isem0)
        idx_start(1, idx1, isem1)
        idx_wait(idx0, isem0)
        gat_start(idx0, rows0, gsem0)
        idx_wait(idx1, isem1)
        gat_start(idx1, rows1, gsem1)
        gat_wait(idx0, rows0, gsem0)
        out_start(0, rows0, osem0)
        idx_start(2, idx0, isem0)
        gat_wait(idx1, rows1, gsem1)
        out_start(1, rows1, osem1)
        idx_start(3, idx1, isem1)

        def body(i, carry):
            c = 2 * i
            idx_wait(idx0, isem0)
            out_wait(rows0, osem0)
            gat_start(idx0, rows0, gsem0)
            idx_wait(idx1, isem1)
            out_wait(rows1, osem1)
            gat_start(idx1, rows1, gsem1)
            gat_wait(idx0, rows0, gsem0)
            out_start(c, rows0, osem0)
            idx_start(c + 2, idx0, isem0)
            gat_wait(idx1, rows1, gsem1)
            out_start(c + 1, rows1, osem1)
            idx_start(c + 3, idx1, isem1)
            return carry

        lax.fori_loop(1, n_pairs - 1, body, 0)

        c = n_chunks - 2
        idx_wait(idx0, isem0)
        out_wait(rows0, osem0)
        gat_start(idx0, rows0, gsem0)
        idx_wait(idx1, isem1)
        out_wait(rows1, osem1)
        gat_start(idx1, rows1, gsem1)
        gat_wait(idx0, rows0, gsem0)
        out_start(c, rows0, osem0)
        gat_wait(idx1, rows1, gsem1)
        out_start(c + 1, rows1, osem1)
        out_wait(rows0, osem0)
        out_wait(rows1, osem1)

    return k(table, idx_flat)


def kernel(x, table):
    n_rows = x.shape[0] * x.shape[1]
    out = _sc_gather(table, x.reshape(n_rows), n_rows)
    return out.reshape(x.shape + (DIM,))
